# baseline (speedup 1.0000x reference)
import functools

import jax
import jax.numpy as jnp
from jax import lax
from jax.experimental import pallas as pl
from jax.experimental.pallas import tpu as pltpu
from jax.experimental.pallas import tpu_sc as plsc

NC = 2
NS = 16
NW = NC * NS
L = 16


@functools.lru_cache(maxsize=None)
def _make_gather(V, D, B, H):
    n_full = V // 128
    tail = V - n_full * 128
    base_cnt = n_full // NW
    extra = n_full - base_cnt * NW
    b_per_w = B // NW
    n_ivec = b_per_w // 128
    mesh = plsc.VectorSubcoreMesh(core_axis_name="c", subcore_axis_name="s")

    @functools.partial(
        pl.kernel,
        out_type=jax.ShapeDtypeStruct((H, D, B), jnp.float32),
        mesh=mesh,
        scratch_types=[
            pltpu.HBM((V * D // 128, 128), jnp.float32),
            pltpu.VMEM((D, 128), jnp.float32),
            pltpu.VMEM((128 * D // 128, 128), jnp.float32),
            pltpu.VMEM((H, b_per_w), jnp.int32),
            pltpu.VMEM((128,), jnp.int32),
            pltpu.VMEM((128, 128), jnp.float32),
            pltpu.VMEM((D, 128), jnp.float32),
            pltpu.VMEM((tail * D,), jnp.float32),
            pltpu.SemaphoreType.DMA,
            pltpu.SemaphoreType.REGULAR,
        ],
        compiler_params=pltpu.CompilerParams(needs_layout_passes=False),
    )
    def k(wt_hbm, xt_hbm, wtail_hbm, out_hbm, wrm, tin, tout, idx_v, idx_l,
          lines, rowsT, tail_f, sem_g, sem_x):
        core = lax.axis_index("c")
        sub = lax.axis_index("s")
        wid = sub * NC + core

        iota_lo = lax.iota(jnp.int32, L)
        iota_hi = iota_lo + L
        lpt = 128 * D // 128

        start = wid * base_cnt + jnp.minimum(wid, extra)
        cnt = jnp.where(wid < extra, base_cnt + 1, base_cnt)

        def tile_body(kk, carry):
            @pl.when(kk < cnt)
            def _():
                t = start + kk
                pltpu.sync_copy(wt_hbm.at[:, pl.ds(t * 128, 128)], tin)
                for l in range(lpt):
                    for m in range(128 // lpt):
                        csp = jnp.full((L,), l * (128 // lpt) + m, jnp.int32)
                        v1 = plsc.load_gather(tin, [iota_lo, csp])
                        v2 = plsc.load_gather(tin, [iota_hi, csp])
                        tout[l, pl.ds(m * D, L)] = v1
                        tout[l, pl.ds(m * D + L, L)] = v2
                pltpu.sync_copy(tout, wrm.at[pl.ds(t * lpt, lpt), :])
            return carry

        lax.fori_loop(0, base_cnt + 1, tile_body, 0)

        if tail:
            @pl.when(wid == extra)
            def _():
                pltpu.sync_copy(wtail_hbm, tail_f)
                for i in range(tail * D // L):
                    v = tail_f[pl.ds(i * L, L)]
                    tout[i // (128 // L), pl.ds((i % (128 // L)) * L, L)] = v
                pltpu.sync_copy(tout.at[pl.ds(0, tail * D // 128), :],
                                wrm.at[pl.ds(n_full * lpt, tail * D // 128), :])

        plsc.subcore_barrier()

        @pl.when(sub == 0)
        def _():
            pl.semaphore_signal(sem_x, 1, core_index=1 - core)
            pl.semaphore_wait(sem_x, 1)

        plsc.subcore_barrier()

        pltpu.sync_copy(xt_hbm.at[:, pl.ds(wid * b_per_w, b_per_w)], idx_v)

        rpl = 128 // D
        n_q = b_per_w // 128

        def h_body(h, carry):
            for bq in range(n_q):
                for g in range(128 // L):
                    iv = idx_v[h, pl.ds(bq * 128 + g * L, L)]
                    idx_l[pl.ds(g * L, L)] = lax.shift_right_logical(iv, 2)
                pltpu.async_copy(wrm.at[idx_l], lines, sem_g).wait()
                for g in range(128 // L):
                    iv = idx_v[h, pl.ds(bq * 128 + g * L, L)]
                    colbase = (iv & (rpl - 1)) * D
                    rvec = iota_lo + g * L
                    for f in range(D):
                        v = plsc.load_gather(lines, [rvec, colbase + f])
                        rowsT[f, pl.ds(g * L, L)] = v
                pltpu.sync_copy(
                    rowsT,
                    out_hbm.at[h, :, pl.ds(wid * b_per_w + bq * 128, 128)])
            return carry

        lax.fori_loop(0, H, h_body, 0)

    return k


def kernel(x, W):
    V, D = W.shape
    B, H = x.shape
    n_full = V // 128
    xt = x.T.astype(jnp.int32)
    wt = W.T
    wtail = W[n_full * 128:].reshape(-1)
    out_t = _make_gather(V, D, B, H)(wt, xt, wtail)
    return out_t.transpose(2, 0, 1)


# speedup vs baseline: 1.7206x; 1.7206x over previous
import functools

import jax
import jax.numpy as jnp
from jax import lax
from jax.experimental import pallas as pl
from jax.experimental.pallas import tpu as pltpu
from jax.experimental.pallas import tpu_sc as plsc

NC = 2
NS = 16
NW = NC * NS
L = 16


@functools.lru_cache(maxsize=None)
def _make_gather(V, D, B, H):
    rpl = 128 // D
    n_lines = V // rpl
    n_full = V // 128
    tail = V - n_full * 128
    CH = 512
    n_chunks = n_full * 128 // CH
    base_cnt = n_chunks // NW
    extra = n_chunks - base_cnt * NW
    lpc = CH // rpl
    b_per_w = B // NW
    n_u = H * (b_per_w // 128)
    HG = 8
    mesh = plsc.VectorSubcoreMesh(core_axis_name="c", subcore_axis_name="s")

    @functools.partial(
        pl.kernel,
        out_type=jax.ShapeDtypeStruct((H, D, B), jnp.float32),
        mesh=mesh,
        scratch_types=[
            pltpu.HBM((n_lines, 128), jnp.float32),
            pltpu.SemaphoreType.DMA,
            pltpu.SemaphoreType.DMA,
            pltpu.SemaphoreType.DMA,
            pltpu.SemaphoreType.DMA,
            pltpu.SemaphoreType.DMA,
            pltpu.SemaphoreType.DMA,
            pltpu.SemaphoreType.REGULAR,
        ],
        compiler_params=pltpu.CompilerParams(needs_layout_passes=False),
    )
    def k(wt_hbm, xt_hbm, wtail_hbm, out_hbm, wrm,
          si0, si1, sw0, sw1, sg0, sg1, sem_x):
        core = lax.axis_index("c")
        sub = lax.axis_index("s")
        wid = sub * NC + core
        sis = (si0, si1)
        sws = (sw0, sw1)
        sgs = (sg0, sg1)

        iota_lo = lax.iota(jnp.int32, L)
        iota_hi = iota_lo + L

        start = wid * base_cnt + jnp.minimum(wid, extra)
        cnt = jnp.where(wid < extra, base_cnt + 1, base_cnt)

        def stage_a(tin0, tin1, tout0, tout1, tail_f):
            tins = (tin0, tin1)
            touts = (tout0, tout1)

            def in_copy(c, b):
                return pltpu.make_async_copy(
                    wt_hbm.at[:, pl.ds((start + c) * CH, CH)], tins[b], sis[b])

            def out_copy(c, b):
                return pltpu.make_async_copy(
                    touts[b], wrm.at[pl.ds((start + c) * lpc, lpc), :], sws[b])

            @pl.when(0 < cnt)
            def _():
                in_copy(0, 0).start()

            @pl.when(1 < cnt)
            def _():
                in_copy(1, 1).start()

            def chunk_body(c, b):
                @pl.when(c < cnt)
                def _():
                    in_copy(c, b).wait()

                    @pl.when(c >= 2)
                    def _():
                        out_copy(c - 2, b).wait()

                    tin, tout = tins[b], touts[b]

                    def l_body(l, carry):
                        for j in range(CH // 128):
                            for m in range(rpl):
                                csp = jnp.full((L,), j * 128 + m, jnp.int32) \
                                    + l * rpl
                                v1 = plsc.load_gather(tin, [iota_lo, csp])
                                v2 = plsc.load_gather(tin, [iota_hi, csp])
                                tout[j * 32 + l, pl.ds(m * D, L)] = v1
                                tout[j * 32 + l, pl.ds(m * D + L, L)] = v2
                        return carry

                    lax.fori_loop(0, 128 // rpl, l_body, 0)
                    out_copy(c, b).start()

                    @pl.when(c + 2 < cnt)
                    def _():
                        in_copy(c + 2, b).start()

            def pair_body(g, carry):
                chunk_body(2 * g, 0)
                chunk_body(2 * g + 1, 1)
                return carry

            lax.fori_loop(0, (base_cnt + 2) // 2, pair_body, 0)

            @pl.when(cnt % 2 == 0)
            def _():
                out_copy(cnt - 2, 0).wait()
                out_copy(cnt - 1, 1).wait()

            @pl.when(cnt % 2 == 1)
            def _():
                out_copy(cnt - 2, 1).wait()
                out_copy(cnt - 1, 0).wait()

            if tail:
                @pl.when(wid == NW - 1)
                def _():
                    pltpu.sync_copy(wtail_hbm, tail_f)
                    for i in range(tail * D // L):
                        v = tail_f[pl.ds(i * L, L)]
                        tout0[i // (128 // L), pl.ds((i % (128 // L)) * L, L)] = v
                    pltpu.sync_copy(
                        tout0.at[pl.ds(0, tail * D // 128), :],
                        wrm.at[pl.ds(n_full * 128 // rpl, tail * D // 128), :])

        pl.run_scoped(
            stage_a,
            pltpu.VMEM((D, CH), jnp.float32),
            pltpu.VMEM((D, CH), jnp.float32),
            pltpu.VMEM((lpc, 128), jnp.float32),
            pltpu.VMEM((lpc, 128), jnp.float32),
            pltpu.VMEM((tail * D,), jnp.float32),
        )

        plsc.subcore_barrier()

        @pl.when(sub == 0)
        def _():
            pl.semaphore_signal(sem_x, 1, core_index=1 - core)
            pl.semaphore_wait(sem_x, 1)

        plsc.subcore_barrier()

        def stage_c(idx_v, il0, il1, ln0, ln1, rt0, rt1):
            ils = (il0, il1)
            lns = (ln0, ln1)
            rts = (rt0, rt1)
            col0 = wid * b_per_w

            def stage_idx(u):
                grp = u // (4 * HG)

                @pl.when(grp < H // HG)
                def _():
                    pltpu.sync_copy(
                        xt_hbm.at[pl.ds(grp * HG, HG), pl.ds(col0, b_per_w)],
                        idx_v)

                if H % HG:
                    @pl.when(grp == H // HG)
                    def _():
                        pltpu.sync_copy(
                            xt_hbm.at[pl.ds((H // HG) * HG, H % HG),
                                      pl.ds(col0, b_per_w)],
                            idx_v.at[pl.ds(0, H % HG), :])

            def prep_lines(u, b):
                hl = (u // 4) % HG
                bq = u % 4
                for g in range(128 // L):
                    iv = idx_v[hl, pl.ds(bq * 128 + g * L, L)]
                    ils[b][pl.ds(g * L, L)] = lax.shift_right_logical(iv, 2)

            def gather_copy(b):
                return pltpu.make_async_copy(wrm.at[ils[b]], lns[b], sgs[b])

            def out_copy(u, b):
                h = u // 4
                bq = u % 4
                return pltpu.make_async_copy(
                    rts[b],
                    out_hbm.at[h, :, pl.ds(col0 + bq * 128, 128)],
                    sws[b])

            stage_idx(0)
            prep_lines(0, 0)
            gather_copy(0).start()

            def unit_body(u, b):
                gather_copy(b).wait()

                @pl.when(u >= 2)
                def _():
                    out_copy(u - 2, b).wait()

                hl = (u // 4) % HG
                bq = u % 4
                lines, rowsT = lns[b], rts[b]

                def g_body(g, carry):
                    iv = idx_v[hl, pl.ds(bq * 128 + g * L, L)]
                    colbase = (iv & (rpl - 1)) * D
                    rvec = iota_lo + g * L
                    for f in range(D):
                        v = plsc.load_gather(lines, [rvec, colbase + f])
                        rowsT[f, pl.ds(g * L, L)] = v
                    return carry

                lax.fori_loop(0, 128 // L, g_body, 0)
                out_copy(u, b).start()

                @pl.when(u + 1 < n_u)
                def _():
                    @pl.when((u + 1) % (4 * HG) == 0)
                    def _():
                        stage_idx(u + 1)
                    prep_lines(u + 1, 1 - b)
                    gather_copy(1 - b).start()

            def pair_body(g, carry):
                unit_body(2 * g, 0)
                unit_body(2 * g + 1, 1)
                return carry

            lax.fori_loop(0, n_u // 2, pair_body, 0)
            out_copy(n_u - 2, 0).wait()
            out_copy(n_u - 1, 1).wait()

        pl.run_scoped(
            stage_c,
            pltpu.VMEM((HG, b_per_w), jnp.int32),
            pltpu.VMEM((128,), jnp.int32),
            pltpu.VMEM((128,), jnp.int32),
            pltpu.VMEM((128, 128), jnp.float32),
            pltpu.VMEM((128, 128), jnp.float32),
            pltpu.VMEM((D, 128), jnp.float32),
            pltpu.VMEM((D, 128), jnp.float32),
        )

    return k


def kernel(x, W):
    V, D = W.shape
    B, H = x.shape
    n_full = V // 128
    xt = x.T.astype(jnp.int32)
    wt = W.T
    wtail = W[n_full * 128:].reshape(-1)
    out_t = _make_gather(V, D, B, H)(wt, xt, wtail)
    return out_t.transpose(2, 0, 1)


# speedup vs baseline: 2.2921x; 1.3321x over previous
import functools

import jax
import jax.numpy as jnp
from jax import lax
from jax.experimental import pallas as pl
from jax.experimental.pallas import tpu as pltpu
from jax.experimental.pallas import tpu_sc as plsc

NC = 2
NS = 16
NW = NC * NS
L = 16


@functools.lru_cache(maxsize=None)
def _make_gather(V, D, B, H):
    rpl = 128 // D
    n_lines = V // rpl
    n_full = V // 128
    tail = V - n_full * 128
    CH = 512
    n_chunks = n_full * 128 // CH
    base_cnt = n_chunks // NW
    extra = n_chunks - base_cnt * NW
    lpc = CH // rpl
    b_per_w = B // NW
    n_u = H * (b_per_w // 128)
    HG = 8
    mesh = plsc.VectorSubcoreMesh(core_axis_name="c", subcore_axis_name="s")

    @functools.partial(
        pl.kernel,
        out_type=jax.ShapeDtypeStruct((H, D, B), jnp.float32),
        mesh=mesh,
        scratch_types=[
            pltpu.HBM((n_lines, 128), jnp.float32),
            pltpu.SemaphoreType.DMA,
            pltpu.SemaphoreType.DMA,
            pltpu.SemaphoreType.DMA,
            pltpu.SemaphoreType.DMA,
            pltpu.SemaphoreType.DMA,
            pltpu.SemaphoreType.DMA,
            pltpu.SemaphoreType.DMA,
            pltpu.SemaphoreType.DMA,
            pltpu.SemaphoreType.REGULAR,
        ],
        compiler_params=pltpu.CompilerParams(needs_layout_passes=False),
    )
    def k(wt_hbm, xt_hbm, wtail_hbm, out_hbm, wrm,
          si0, si1, sw0, sw1, sg0, sg1, sg2, sg3, sem_x):
        core = lax.axis_index("c")
        sub = lax.axis_index("s")
        wid = sub * NC + core
        sis = (si0, si1)
        sws = (sw0, sw1)
        sgs = (sg0, sg1, sg2, sg3)

        iota_lo = lax.iota(jnp.int32, L)
        iota_hi = iota_lo + L

        start = wid * base_cnt + jnp.minimum(wid, extra)
        cnt = jnp.where(wid < extra, base_cnt + 1, base_cnt)

        def stage_a(tin0, tin1, tout0, tout1, tail_f):
            tins = (tin0, tin1)
            touts = (tout0, tout1)

            def in_copy(c, b):
                return pltpu.make_async_copy(
                    wt_hbm.at[:, pl.ds((start + c) * CH, CH)], tins[b], sis[b])

            def out_copy(c, b):
                return pltpu.make_async_copy(
                    touts[b], wrm.at[pl.ds((start + c) * lpc, lpc), :], sws[b])

            @pl.when(0 < cnt)
            def _():
                in_copy(0, 0).start()

            @pl.when(1 < cnt)
            def _():
                in_copy(1, 1).start()

            def chunk_body(c, b):
                @pl.when(c < cnt)
                def _():
                    in_copy(c, b).wait()

                    @pl.when(c >= 2)
                    def _():
                        out_copy(c - 2, b).wait()

                    tin, tout = tins[b], touts[b]

                    def l_body(l, carry):
                        for j in range(CH // 128):
                            for m in range(rpl):
                                csp = jnp.full((L,), j * 128 + m, jnp.int32) \
                                    + l * rpl
                                v1 = plsc.load_gather(tin, [iota_lo, csp])
                                v2 = plsc.load_gather(tin, [iota_hi, csp])
                                tout[j * 32 + l, pl.ds(m * D, L)] = v1
                                tout[j * 32 + l, pl.ds(m * D + L, L)] = v2
                        return carry

                    lax.fori_loop(0, 128 // rpl, l_body, 0)
                    out_copy(c, b).start()

                    @pl.when(c + 2 < cnt)
                    def _():
                        in_copy(c + 2, b).start()

            def pair_body(g, carry):
                chunk_body(2 * g, 0)
                chunk_body(2 * g + 1, 1)
                return carry

            lax.fori_loop(0, (base_cnt + 2) // 2, pair_body, 0)

            @pl.when(cnt % 2 == 0)
            def _():
                out_copy(cnt - 2, 0).wait()
                out_copy(cnt - 1, 1).wait()

            @pl.when(cnt % 2 == 1)
            def _():
                out_copy(cnt - 2, 1).wait()
                out_copy(cnt - 1, 0).wait()

            if tail:
                @pl.when(wid == NW - 1)
                def _():
                    pltpu.sync_copy(wtail_hbm, tail_f)
                    for i in range(tail * D // L):
                        v = tail_f[pl.ds(i * L, L)]
                        tout0[i // (128 // L), pl.ds((i % (128 // L)) * L, L)] = v
                    pltpu.sync_copy(
                        tout0.at[pl.ds(0, tail * D // 128), :],
                        wrm.at[pl.ds(n_full * 128 // rpl, tail * D // 128), :])

        pl.run_scoped(
            stage_a,
            pltpu.VMEM((D, CH), jnp.float32),
            pltpu.VMEM((D, CH), jnp.float32),
            pltpu.VMEM((lpc, 128), jnp.float32),
            pltpu.VMEM((lpc, 128), jnp.float32),
            pltpu.VMEM((tail * D,), jnp.float32),
        )

        plsc.subcore_barrier()

        @pl.when(sub == 0)
        def _():
            pl.semaphore_signal(sem_x, 1, core_index=1 - core)
            pl.semaphore_wait(sem_x, 1)

        plsc.subcore_barrier()

        NB = 4

        def stage_c(idx_v, il0, il1, il2, il3, im0, im1, im2, im3,
                    ln0, ln1, ln2, ln3, rt0, rt1):
            ils = (il0, il1, il2, il3)
            ims = (im0, im1, im2, im3)
            lns = (ln0, ln1, ln2, ln3)
            rts = (rt0, rt1)
            col0 = wid * b_per_w

            def stage_idx(u):
                grp = u // (4 * HG)

                @pl.when(grp < H // HG)
                def _():
                    pltpu.sync_copy(
                        xt_hbm.at[pl.ds(grp * HG, HG), pl.ds(col0, b_per_w)],
                        idx_v)

                if H % HG:
                    @pl.when(grp == H // HG)
                    def _():
                        pltpu.sync_copy(
                            xt_hbm.at[pl.ds((H // HG) * HG, H % HG),
                                      pl.ds(col0, b_per_w)],
                            idx_v.at[pl.ds(0, H % HG), :])

            def prep_lines(u, b):
                hl = (u // 4) % HG
                bq = u % 4
                for g in range(128 // L):
                    iv = idx_v[hl, pl.ds(bq * 128 + g * L, L)]
                    ils[b][pl.ds(g * L, L)] = lax.shift_right_logical(iv, 2)
                    ims[b][pl.ds(g * L, L)] = iv & (rpl - 1)

            def gather_copy(b):
                return pltpu.make_async_copy(wrm.at[ils[b]], lns[b], sgs[b])

            def out_copy(u, br):
                h = u // 4
                bq = u % 4
                return pltpu.make_async_copy(
                    rts[br],
                    out_hbm.at[h, :, pl.ds(col0 + bq * 128, 128)],
                    sws[br])

            stage_idx(0)
            for p in range(NB - 1):
                prep_lines(p, p)
                gather_copy(p).start()

            def unit_body(u, b):
                br = b % 2
                gather_copy(b).wait()

                @pl.when(u + NB - 1 < n_u)
                def _():
                    @pl.when((u + NB - 1) % (4 * HG) == 0)
                    def _():
                        stage_idx(u + NB - 1)
                    prep_lines(u + NB - 1, (b + NB - 1) % NB)
                    gather_copy((b + NB - 1) % NB).start()

                @pl.when(u >= 2)
                def _():
                    out_copy(u - 2, br).wait()

                lines, rowsT = lns[b], rts[br]
                im = ims[b]

                def g_body(g, carry):
                    colbase = im[pl.ds(g * L, L)] * D
                    rvec = iota_lo + g * L
                    for f in range(D):
                        v = plsc.load_gather(lines, [rvec, colbase + f])
                        rowsT[f, pl.ds(g * L, L)] = v
                    return carry

                lax.fori_loop(0, 128 // L, g_body, 0)
                out_copy(u, br).start()

            def quad_body(g, carry):
                for qu in range(NB):
                    unit_body(NB * g + qu, qu)
                return carry

            lax.fori_loop(0, n_u // NB, quad_body, 0)
            out_copy(n_u - 2, 0).wait()
            out_copy(n_u - 1, 1).wait()

        pl.run_scoped(
            stage_c,
            pltpu.VMEM((HG, b_per_w), jnp.int32),
            pltpu.VMEM((128,), jnp.int32),
            pltpu.VMEM((128,), jnp.int32),
            pltpu.VMEM((128,), jnp.int32),
            pltpu.VMEM((128,), jnp.int32),
            pltpu.VMEM((128,), jnp.int32),
            pltpu.VMEM((128,), jnp.int32),
            pltpu.VMEM((128,), jnp.int32),
            pltpu.VMEM((128,), jnp.int32),
            pltpu.VMEM((128, 128), jnp.float32),
            pltpu.VMEM((128, 128), jnp.float32),
            pltpu.VMEM((128, 128), jnp.float32),
            pltpu.VMEM((128, 128), jnp.float32),
            pltpu.VMEM((D, 128), jnp.float32),
            pltpu.VMEM((D, 128), jnp.float32),
        )

    return k


def kernel(x, W):
    V, D = W.shape
    B, H = x.shape
    n_full = V // 128
    xt = x.T.astype(jnp.int32)
    wt = W.T
    wtail = W[n_full * 128:].reshape(-1)
    out_t = _make_gather(V, D, B, H)(wt, xt, wtail)
    return out_t.transpose(2, 0, 1)
